# baseline (speedup 1.0000x reference)
import functools

import jax
import jax.numpy as jnp
from jax import lax
from jax.experimental import pallas as pl
from jax.experimental.pallas import tpu as pltpu
from jax.experimental.pallas import tpu_sc as plsc

_VOCAB = 1000000
_D = 32
_B = 4096
_L = 200

_NC = 2
_NS = 16
_LN = 16
_NW = _NC * _NS
_ROWS = _B // _NW
_NG = (_L + _LN - 1) // _LN
_EMB_ROWS = _NG * _LN
_C1 = 104
_C2OFF = 96
_DEPTH = 2
_QW = 128


_NFULL = _VOCAB // _QW * _QW
_NBLK = _NFULL // _QW
_TAILW = _VOCAB - _NFULL


def _make_transpose_kernel():
    mesh = plsc.VectorSubcoreMesh(core_axis_name="c", subcore_axis_name="s")

    @functools.partial(
        pl.kernel,
        mesh=mesh,
        compiler_params=pltpu.CompilerParams(
            needs_layout_passes=False, use_tc_tiling_on_sc=True),
        out_type=jax.ShapeDtypeStruct((_VOCAB // 4, _QW), jnp.float32),
        scratch_types=[
            [pltpu.VMEM((_D, _QW), jnp.float32) for _ in range(2)],
            [pltpu.VMEM((_D, _QW), jnp.float32) for _ in range(2)],
            [pltpu.SemaphoreType.DMA for _ in range(2)],
            [pltpu.SemaphoreType.DMA for _ in range(2)],
        ],
    )
    def t_kernel(tabt_hbm, tail_hbm, out_hbm, ibs, obs, sis, sos):
        wid = lax.axis_index("s") * _NC + lax.axis_index("c")
        iota = lax.iota(jnp.int32, _LN)
        iota_q = iota >> 2
        iotalo = (iota & 3) * _D

        def in_descs(k, ib, sem):
            c0 = (wid + _NW * k) * _QW
            return [pltpu.make_async_copy(
                tabt_hbm.at[pl.ds(8 * rr, 8), pl.ds(c0, _QW)],
                ib.at[pl.ds(8 * rr, 8)], sem) for rr in range(4)]

        def blk_ok(k):
            return wid + _NW * k < _NBLK

        def start_in(k, ib, sem):
            @pl.when(blk_ok(k))
            def _():
                for dsc in in_descs(k, ib, sem):
                    dsc.start()

        def wait_in(k, ib, sem):
            @pl.when(blk_ok(k))
            def _():
                for dsc in in_descs(k, ib, sem):
                    dsc.wait()

        def out_desc(k, ob, sem):
            return pltpu.make_async_copy(
                ob, out_hbm.at[pl.ds((wid + _NW * k) * _D, _D)], sem)

        def transpose_blk(k, ib, ob, sem):
            @pl.when(blk_ok(k))
            def _():
                for d in range(_D):
                    dvec = jnp.full((_LN,), d, jnp.int32)
                    for vg in range(_QW // _LN):
                        val = plsc.load_gather(ib, [dvec, vg * _LN + iota])
                        plsc.store_scatter(
                            ob, [4 * vg + iota_q, d + iotalo], val)
                out_desc(k, ob, sem).start()

        def wait_out(k, ob, sem):
            @pl.when((k >= 0) & blk_ok(k))
            def _():
                out_desc(k, ob, sem).wait()

        start_in(0, ibs[0], sis[0])

        def pair_body(q, _):
            for j in range(2):
                k = 2 * q + j
                start_in(k + 1, ibs[1 - j], sis[1 - j])
                wait_in(k, ibs[j], sis[j])
                wait_out(k - 2, obs[j], sos[j])
                transpose_blk(k, ibs[j], obs[j], sos[j])
            return 0

        npairs = (_NBLK // _NW + 2) // 2
        lax.fori_loop(0, npairs, pair_body, 0)
        wait_out(2 * npairs - 2, obs[0], sos[0])
        wait_out(2 * npairs - 1, obs[1], sos[1])

        @pl.when(wid == 0)
        def _():
            pltpu.sync_copy(tail_hbm, ibs[0].at[pl.ds(0, _TAILW // 4)])
            pltpu.sync_copy(ibs[0].at[pl.ds(0, _TAILW // 4)],
                            out_hbm.at[pl.ds(_NFULL // 4, _TAILW // 4)])

    return t_kernel


def _make_sc_kernel():
    mesh = plsc.VectorSubcoreMesh(core_axis_name="c", subcore_axis_name="s")

    @functools.partial(
        pl.kernel,
        mesh=mesh,
        compiler_params=pltpu.CompilerParams(
            needs_layout_passes=False, use_tc_tiling_on_sc=True),
        out_type=jax.ShapeDtypeStruct((_B,), jnp.float32),
        scratch_types=[
            pltpu.VMEM((_ROWS * _L,), jnp.int32),
            [pltpu.VMEM((_L + 8, _QW), jnp.float32)
             for _ in range(_DEPTH)],
            [pltpu.VMEM((_L + 8,), jnp.int32)
             for _ in range(_DEPTH)],
            pltpu.VMEM((_ROWS,), jnp.int32),
            pltpu.VMEM((48,), jnp.float32),
            pltpu.VMEM((_ROWS,), jnp.float32),
            [pltpu.SemaphoreType.DMA for _ in range(_DEPTH)],
        ],
    )
    def sc_kernel(x_hbm, lens_hbm, table_hbm, wb_hbm, out_hbm,
                  xv, embs, idxs, lens_v, wb_v, scores, sems):
        wid = lax.axis_index("s") * _NC + lax.axis_index("c")
        base = wid * _ROWS

        pltpu.sync_copy(x_hbm.at[pl.ds(base * _L, _ROWS * _L)], xv)
        pltpu.sync_copy(lens_hbm.at[pl.ds(base, _ROWS)], lens_v)
        pltpu.sync_copy(wb_hbm, wb_v)
        wv0 = wb_v[pl.ds(0, _LN)]
        wv1 = wb_v[pl.ds(_LN, _LN)]
        wv2 = wb_v[pl.ds(2 * _LN, _LN)]
        w_d = [wv0[d] for d in range(_LN)] + [wv1[d] for d in range(_LN)]
        b_vec = jnp.full((_LN,), wv2[0], jnp.float32)
        iota = lax.iota(jnp.int32, _LN)
        zero16 = jnp.zeros((_LN,), jnp.float32)

        def row_len(r):
            return plsc.load_gather(lens_v, [jnp.full((_LN,), r, jnp.int32)])

        def gather_descs(r, emb, idxb, sem):
            d1 = pltpu.make_async_copy(
                table_hbm.at[idxb.at[pl.ds(0, _C1)]],
                emb.at[pl.ds(0, _C1)], sem)
            d2 = pltpu.make_async_copy(
                table_hbm.at[idxb.at[pl.ds(_C2OFF, _C1)]],
                emb.at[pl.ds(_C2OFF, _C1)], sem)
            return d1, d2

        def start_row(r, emb, idxb, sem, ln):
            rbase = r * _L + iota
            for g in range(12):
                off = g * _LN
                v = plsc.load_gather(xv, [rbase + off])
                idxb[pl.ds(off, _LN)] = v >> 2
            v = plsc.load_gather(xv, [rbase + (_L - _LN)])
            idxb[pl.ds(_L - _LN, _LN)] = v >> 2
            d1, d2 = gather_descs(r, emb, idxb, sem)
            d1.start()

            @pl.when(ln > _C2OFF)
            def _():
                d2.start()

        def wait_row(r, emb, idxb, sem, ln):
            d1, d2 = gather_descs(r, emb, idxb, sem)
            d1.wait()

            @pl.when(ln > _C2OFF)
            def _():
                d2.wait()

        def compute_row(r, emb, len_vec):
            ng = (len_vec[0] + (_LN - 1)) >> 4

            def grp_body(g, carry):
                s_acc, t_acc, wt_acc = carry
                tok = g * _LN + iota
                mask = tok < len_vec
                vtok = plsc.load_gather(
                    xv, [r * _L + jnp.minimum(tok, _L - 1)])
                rem32 = (vtok & 3) * _D
                norm = zero16
                p = zero16
                for d in range(_D):
                    a = plsc.load_gather(emb, [tok, rem32 + d])
                    norm = norm + a * a
                    am = jnp.where(mask, a, 0.0)
                    p = p + w_d[d] * am
                w = jnp.where(mask, jnp.exp(norm), 0.0)
                return (s_acc + p, t_acc + w * p, wt_acc + w)

            s_acc, t_acc, wt_acc = lax.fori_loop(
                0, ng, grp_body, (zero16, zero16, zero16))
            lenf_vec = len_vec.astype(jnp.float32)
            s_vec = jnp.full((_LN,), jnp.sum(s_acc), jnp.float32)
            t_vec = jnp.full((_LN,), jnp.sum(t_acc), jnp.float32)
            w_vec = jnp.full((_LN,), jnp.sum(wt_acc), jnp.float32)
            score_vec = s_vec / lenf_vec + t_vec / w_vec + b_vec
            plsc.store_scatter(scores, [jnp.full((_LN,), r, jnp.int32)],
                               score_vec, mask=iota == 0)

        for j in range(_DEPTH - 1):
            lv = row_len(j)
            start_row(j, embs[j], idxs[j], sems[j], lv[0])

        def quad_body(q, _):
            r0 = _DEPTH * q
            for j in range(_DEPTH):
                r = r0 + j
                lv = row_len(r)
                wait_row(r, embs[j], idxs[j], sems[j], lv[0])
                rn = r + _DEPTH - 1
                jn = (j + _DEPTH - 1) % _DEPTH

                @pl.when(rn < _ROWS)
                def _(rn=rn, jn=jn):
                    lvn = row_len(rn)
                    start_row(rn, embs[jn], idxs[jn], sems[jn], lvn[0])

                compute_row(r, embs[j], lv)
            return 0

        lax.fori_loop(0, _ROWS // _DEPTH, quad_body, 0)

        for g in range(_ROWS // _LN):
            sv = scores[pl.ds(g * _LN, _LN)]
            scores[pl.ds(g * _LN, _LN)] = 1.0 / (1.0 + jnp.exp(-sv))
        pltpu.sync_copy(scores, out_hbm.at[pl.ds(base, _ROWS)])

    return sc_kernel


_transpose_kernel = _make_transpose_kernel()
_sc_kernel = _make_sc_kernel()


def kernel(X, lens, table, W, b):
    wb = jnp.concatenate(
        [W.reshape(-1).astype(jnp.float32),
         b.reshape(-1).astype(jnp.float32),
         jnp.zeros((48 - _D - 1,), jnp.float32)])
    tail = table[_NFULL:, :].reshape(_TAILW // 4, 4 * _D)
    lin = _transpose_kernel(table.T, tail)
    prob = _sc_kernel(X.astype(jnp.int32).reshape(-1),
                      lens.astype(jnp.int32), lin, wb)
    return prob.reshape(_B, 1)


# speedup vs baseline: 1.3768x; 1.3768x over previous
import functools

import jax
import jax.numpy as jnp
from jax import lax
from jax.experimental import pallas as pl
from jax.experimental.pallas import tpu as pltpu
from jax.experimental.pallas import tpu_sc as plsc

_VOCAB = 1000000
_D = 32
_B = 4096
_L = 200

_NC = 2
_NS = 16
_LN = 16
_NW = _NC * _NS
_ROWS = _B // _NW
_NG = (_L + _LN - 1) // _LN
_EMB_ROWS = _NG * _LN
_C1 = 104
_C2OFF = 96
_DEPTH = 2
_QW = 128


_NFULL = _VOCAB // _QW * _QW
_NBLK = _NFULL // _QW
_TAILW = _VOCAB - _NFULL


def _make_transpose_kernel():
    mesh = plsc.VectorSubcoreMesh(core_axis_name="c", subcore_axis_name="s")

    @functools.partial(
        pl.kernel,
        mesh=mesh,
        compiler_params=pltpu.CompilerParams(
            needs_layout_passes=False, use_tc_tiling_on_sc=True),
        out_type=jax.ShapeDtypeStruct((_VOCAB // 4, _QW), jnp.float32),
        scratch_types=[
            [pltpu.VMEM((_D, _QW), jnp.float32) for _ in range(2)],
            [pltpu.VMEM((_D, _QW), jnp.float32) for _ in range(2)],
            [pltpu.SemaphoreType.DMA for _ in range(2)],
            [pltpu.SemaphoreType.DMA for _ in range(2)],
        ],
    )
    def t_kernel(tabt_hbm, tail_hbm, out_hbm, ibs, obs, sis, sos):
        wid = lax.axis_index("s") * _NC + lax.axis_index("c")
        iota = lax.iota(jnp.int32, _LN)
        iota_q = iota >> 2
        iotalo = (iota & 3) * _D

        def in_descs(k, ib, sem):
            c0 = (wid + _NW * k) * _QW
            return [pltpu.make_async_copy(
                tabt_hbm.at[pl.ds(8 * rr, 8), pl.ds(c0, _QW)],
                ib.at[pl.ds(8 * rr, 8)], sem) for rr in range(4)]

        def blk_ok(k):
            return wid + _NW * k < _NBLK

        def start_in(k, ib, sem):
            @pl.when(blk_ok(k))
            def _():
                for dsc in in_descs(k, ib, sem):
                    dsc.start()

        def wait_in(k, ib, sem):
            @pl.when(blk_ok(k))
            def _():
                for dsc in in_descs(k, ib, sem):
                    dsc.wait()

        def out_desc(k, ob, sem):
            return pltpu.make_async_copy(
                ob, out_hbm.at[pl.ds((wid + _NW * k) * _D, _D)], sem)

        def transpose_blk(k, ib, ob, sem):
            @pl.when(blk_ok(k))
            def _():
                for d in range(_D):
                    rowv = (d + iota) & (_D - 1)
                    for c0 in range(0, _QW, _LN):
                        colv = c0 + iota
                        val = plsc.load_gather(ib, [rowv, colv])
                        oflat = colv * _D + rowv
                        plsc.store_scatter(
                            ob, [oflat >> 7, oflat & (_QW - 1)], val)
                out_desc(k, ob, sem).start()

        def wait_out(k, ob, sem):
            @pl.when((k >= 0) & blk_ok(k))
            def _():
                out_desc(k, ob, sem).wait()

        start_in(0, ibs[0], sis[0])

        def pair_body(q, _):
            for j in range(2):
                k = 2 * q + j
                start_in(k + 1, ibs[1 - j], sis[1 - j])
                wait_in(k, ibs[j], sis[j])
                wait_out(k - 2, obs[j], sos[j])
                transpose_blk(k, ibs[j], obs[j], sos[j])
            return 0

        npairs = (_NBLK // _NW + 2) // 2
        lax.fori_loop(0, npairs, pair_body, 0)
        wait_out(2 * npairs - 2, obs[0], sos[0])
        wait_out(2 * npairs - 1, obs[1], sos[1])

        @pl.when(wid == 0)
        def _():
            pltpu.sync_copy(tail_hbm, ibs[0].at[pl.ds(0, _TAILW // 4)])
            pltpu.sync_copy(ibs[0].at[pl.ds(0, _TAILW // 4)],
                            out_hbm.at[pl.ds(_NFULL // 4, _TAILW // 4)])

    return t_kernel


def _make_sc_kernel():
    mesh = plsc.VectorSubcoreMesh(core_axis_name="c", subcore_axis_name="s")

    @functools.partial(
        pl.kernel,
        mesh=mesh,
        compiler_params=pltpu.CompilerParams(
            needs_layout_passes=False, use_tc_tiling_on_sc=True),
        out_type=jax.ShapeDtypeStruct((_B,), jnp.float32),
        scratch_types=[
            pltpu.VMEM((_ROWS * _L,), jnp.int32),
            [pltpu.VMEM((_L + 8, _QW), jnp.float32)
             for _ in range(_DEPTH)],
            [pltpu.VMEM((_L + 8,), jnp.int32)
             for _ in range(_DEPTH)],
            pltpu.VMEM((_ROWS,), jnp.int32),
            pltpu.VMEM((48,), jnp.float32),
            pltpu.VMEM((_ROWS,), jnp.float32),
            [pltpu.SemaphoreType.DMA for _ in range(_DEPTH)],
        ],
    )
    def sc_kernel(x_hbm, lens_hbm, table_hbm, wb_hbm, out_hbm,
                  xv, embs, idxs, lens_v, wb_v, scores, sems):
        wid = lax.axis_index("s") * _NC + lax.axis_index("c")
        base = wid * _ROWS

        pltpu.sync_copy(x_hbm.at[pl.ds(base * _L, _ROWS * _L)], xv)
        pltpu.sync_copy(lens_hbm.at[pl.ds(base, _ROWS)], lens_v)
        pltpu.sync_copy(wb_hbm, wb_v)
        wv0 = wb_v[pl.ds(0, _LN)]
        wv1 = wb_v[pl.ds(_LN, _LN)]
        wv2 = wb_v[pl.ds(2 * _LN, _LN)]
        w_d = [wv0[d] for d in range(_LN)] + [wv1[d] for d in range(_LN)]
        b_vec = jnp.full((_LN,), wv2[0], jnp.float32)
        iota = lax.iota(jnp.int32, _LN)
        zero16 = jnp.zeros((_LN,), jnp.float32)

        def row_len(r):
            return plsc.load_gather(lens_v, [jnp.full((_LN,), r, jnp.int32)])

        def gather_descs(r, emb, idxb, sem):
            d1 = pltpu.make_async_copy(
                table_hbm.at[idxb.at[pl.ds(0, _C1)]],
                emb.at[pl.ds(0, _C1)], sem)
            d2 = pltpu.make_async_copy(
                table_hbm.at[idxb.at[pl.ds(_C2OFF, _C1)]],
                emb.at[pl.ds(_C2OFF, _C1)], sem)
            return d1, d2

        def start_row(r, emb, idxb, sem, ln):
            rbase = r * _L + iota
            for g in range(12):
                off = g * _LN
                v = plsc.load_gather(xv, [rbase + off])
                idxb[pl.ds(off, _LN)] = v >> 2
            v = plsc.load_gather(xv, [rbase + (_L - _LN)])
            idxb[pl.ds(_L - _LN, _LN)] = v >> 2
            d1, d2 = gather_descs(r, emb, idxb, sem)
            d1.start()

            @pl.when(ln > _C2OFF)
            def _():
                d2.start()

        def wait_row(r, emb, idxb, sem, ln):
            d1, d2 = gather_descs(r, emb, idxb, sem)
            d1.wait()

            @pl.when(ln > _C2OFF)
            def _():
                d2.wait()

        def compute_row(r, emb, len_vec):
            ng = (len_vec[0] + (_LN - 1)) >> 4

            def grp_body(g, carry):
                s_acc, t_acc, wt_acc = carry
                tok = g * _LN + iota
                mask = tok < len_vec
                vtok = plsc.load_gather(
                    xv, [r * _L + jnp.minimum(tok, _L - 1)])
                rem32 = (vtok & 3) * _D
                norm = zero16
                p = zero16
                for d in range(_D):
                    a = plsc.load_gather(emb, [tok, rem32 + d])
                    norm = norm + a * a
                    am = jnp.where(mask, a, 0.0)
                    p = p + w_d[d] * am
                w = jnp.where(mask, jnp.exp(norm), 0.0)
                return (s_acc + p, t_acc + w * p, wt_acc + w)

            s_acc, t_acc, wt_acc = lax.fori_loop(
                0, ng, grp_body, (zero16, zero16, zero16))
            lenf_vec = len_vec.astype(jnp.float32)
            s_vec = jnp.full((_LN,), jnp.sum(s_acc), jnp.float32)
            t_vec = jnp.full((_LN,), jnp.sum(t_acc), jnp.float32)
            w_vec = jnp.full((_LN,), jnp.sum(wt_acc), jnp.float32)
            score_vec = s_vec / lenf_vec + t_vec / w_vec + b_vec
            plsc.store_scatter(scores, [jnp.full((_LN,), r, jnp.int32)],
                               score_vec, mask=iota == 0)

        for j in range(_DEPTH - 1):
            lv = row_len(j)
            start_row(j, embs[j], idxs[j], sems[j], lv[0])

        def quad_body(q, _):
            r0 = _DEPTH * q
            for j in range(_DEPTH):
                r = r0 + j
                lv = row_len(r)
                wait_row(r, embs[j], idxs[j], sems[j], lv[0])
                rn = r + _DEPTH - 1
                jn = (j + _DEPTH - 1) % _DEPTH

                @pl.when(rn < _ROWS)
                def _(rn=rn, jn=jn):
                    lvn = row_len(rn)
                    start_row(rn, embs[jn], idxs[jn], sems[jn], lvn[0])

                compute_row(r, embs[j], lv)
            return 0

        lax.fori_loop(0, _ROWS // _DEPTH, quad_body, 0)

        for g in range(_ROWS // _LN):
            sv = scores[pl.ds(g * _LN, _LN)]
            scores[pl.ds(g * _LN, _LN)] = 1.0 / (1.0 + jnp.exp(-sv))
        pltpu.sync_copy(scores, out_hbm.at[pl.ds(base, _ROWS)])

    return sc_kernel


_transpose_kernel = _make_transpose_kernel()
_sc_kernel = _make_sc_kernel()


def kernel(X, lens, table, W, b):
    wb = jnp.concatenate(
        [W.reshape(-1).astype(jnp.float32),
         b.reshape(-1).astype(jnp.float32),
         jnp.zeros((48 - _D - 1,), jnp.float32)])
    tail = table[_NFULL:, :].reshape(_TAILW // 4, 4 * _D)
    lin = _transpose_kernel(table.T, tail)
    prob = _sc_kernel(X.astype(jnp.int32).reshape(-1),
                      lens.astype(jnp.int32), lin, wb)
    return prob.reshape(_B, 1)


# speedup vs baseline: 1.4840x; 1.0778x over previous
import functools

import jax
import jax.numpy as jnp
from jax import lax
from jax.experimental import pallas as pl
from jax.experimental.pallas import tpu as pltpu
from jax.experimental.pallas import tpu_sc as plsc

_VOCAB = 1000000
_D = 32
_B = 4096
_L = 200

_NC = 2
_NS = 16
_LN = 16
_NW = _NC * _NS
_ROWS = _B // _NW
_NG = (_L + _LN - 1) // _LN
_EMB_ROWS = _NG * _LN
_C1 = 104
_C2OFF = 96
_DEPTH = 4
_QW = 128


_NFULL = _VOCAB // _QW * _QW
_NBLK = _NFULL // _QW
_TAILW = _VOCAB - _NFULL


def _make_transpose_kernel():
    mesh = plsc.VectorSubcoreMesh(core_axis_name="c", subcore_axis_name="s")

    @functools.partial(
        pl.kernel,
        mesh=mesh,
        compiler_params=pltpu.CompilerParams(
            needs_layout_passes=False, use_tc_tiling_on_sc=True),
        out_type=jax.ShapeDtypeStruct((_VOCAB // 4, _QW), jnp.float32),
        scratch_types=[
            [pltpu.VMEM((_D, _QW), jnp.float32) for _ in range(4)],
            [pltpu.VMEM((_D, _QW), jnp.float32) for _ in range(4)],
            [pltpu.SemaphoreType.DMA for _ in range(4)],
            [pltpu.SemaphoreType.DMA for _ in range(4)],
        ],
    )
    def t_kernel(tabt_hbm, tail_hbm, out_hbm, ibs, obs, sis, sos):
        wid = lax.axis_index("s") * _NC + lax.axis_index("c")
        iota = lax.iota(jnp.int32, _LN)
        iota_q = iota >> 2
        iotalo = (iota & 3) * _D

        def in_descs(k, ib, sem):
            c0 = (wid + _NW * k) * _QW
            return [pltpu.make_async_copy(
                tabt_hbm.at[pl.ds(8 * rr, 8), pl.ds(c0, _QW)],
                ib.at[pl.ds(8 * rr, 8)], sem) for rr in range(4)]

        def blk_ok(k):
            return wid + _NW * k < _NBLK

        def start_in(k, ib, sem):
            @pl.when(blk_ok(k))
            def _():
                for dsc in in_descs(k, ib, sem):
                    dsc.start()

        def wait_in(k, ib, sem):
            @pl.when(blk_ok(k))
            def _():
                for dsc in in_descs(k, ib, sem):
                    dsc.wait()

        def out_desc(k, ob, sem):
            return pltpu.make_async_copy(
                ob, out_hbm.at[pl.ds((wid + _NW * k) * _D, _D)], sem)

        def transpose_blk(k, ib, ob, sem):
            @pl.when(blk_ok(k))
            def _():
                for d in range(_D):
                    rowv = (d + iota) & (_D - 1)
                    for c0 in range(0, _QW, _LN):
                        colv = c0 + iota
                        val = plsc.load_gather(ib, [rowv, colv])
                        oflat = colv * _D + rowv
                        plsc.store_scatter(
                            ob, [oflat >> 7, oflat & (_QW - 1)], val)
                out_desc(k, ob, sem).start()

        def wait_out(k, ob, sem):
            @pl.when((k >= 0) & blk_ok(k))
            def _():
                out_desc(k, ob, sem).wait()

        for j in range(3):
            start_in(j, ibs[j], sis[j])

        def quad_body(q, _):
            for j in range(4):
                k = 4 * q + j
                jn = (j + 3) % 4
                start_in(k + 3, ibs[jn], sis[jn])
                wait_in(k, ibs[j], sis[j])
                wait_out(k - 4, obs[j], sos[j])
                transpose_blk(k, ibs[j], obs[j], sos[j])
            return 0

        nquads = (_NBLK // _NW + 4) // 4
        lax.fori_loop(0, nquads, quad_body, 0)
        for j in range(4):
            wait_out(4 * (nquads - 1) + j, obs[j], sos[j])

        @pl.when(wid == 0)
        def _():
            pltpu.sync_copy(tail_hbm, ibs[0].at[pl.ds(0, _TAILW // 4)])
            pltpu.sync_copy(ibs[0].at[pl.ds(0, _TAILW // 4)],
                            out_hbm.at[pl.ds(_NFULL // 4, _TAILW // 4)])

    return t_kernel


def _make_sc_kernel():
    mesh = plsc.VectorSubcoreMesh(core_axis_name="c", subcore_axis_name="s")

    @functools.partial(
        pl.kernel,
        mesh=mesh,
        compiler_params=pltpu.CompilerParams(
            needs_layout_passes=False, use_tc_tiling_on_sc=False),
        out_type=jax.ShapeDtypeStruct((_B,), jnp.float32),
        scratch_types=[
            pltpu.VMEM((_ROWS * _L,), jnp.int32),
            [pltpu.VMEM((_L + 8, _D), jnp.float32)
             for _ in range(_DEPTH)],
            [pltpu.VMEM((_L + 8,), jnp.int32)
             for _ in range(_DEPTH)],
            pltpu.VMEM((_ROWS,), jnp.int32),
            pltpu.VMEM((48,), jnp.float32),
            pltpu.VMEM((_ROWS,), jnp.float32),
            [pltpu.SemaphoreType.DMA for _ in range(_DEPTH)],
        ],
    )
    def sc_kernel(x_hbm, lens_hbm, table_hbm, wb_hbm, out_hbm,
                  xv, embs, idxs, lens_v, wb_v, scores, sems):
        wid = lax.axis_index("s") * _NC + lax.axis_index("c")
        base = wid * _ROWS

        pltpu.sync_copy(x_hbm.at[pl.ds(base * _L, _ROWS * _L)], xv)
        pltpu.sync_copy(lens_hbm.at[pl.ds(base, _ROWS)], lens_v)
        pltpu.sync_copy(wb_hbm, wb_v)
        wv2 = wb_v[pl.ds(2 * _LN, _LN)]
        b_vec = jnp.full((_LN,), wv2[0], jnp.float32)
        iota = lax.iota(jnp.int32, _LN)
        zero16 = jnp.zeros((_LN,), jnp.float32)
        wrot = [plsc.load_gather(wb_v, [(d0 + iota) & (_D - 1)])
                for d0 in range(_D)]

        def row_len(r):
            return plsc.load_gather(lens_v, [jnp.full((_LN,), r, jnp.int32)])

        def gather_descs(r, emb, idxb, sem):
            d1 = pltpu.make_async_copy(
                table_hbm.at[idxb.at[pl.ds(0, _C1)]],
                emb.at[pl.ds(0, _C1)], sem)
            d2 = pltpu.make_async_copy(
                table_hbm.at[idxb.at[pl.ds(_C2OFF, _C1)]],
                emb.at[pl.ds(_C2OFF, _C1)], sem)
            return d1, d2

        def start_row(r, emb, idxb, sem, ln):
            rbase = r * _L + iota
            for g in range(12):
                off = g * _LN
                v = plsc.load_gather(xv, [rbase + off])
                idxb[pl.ds(off, _LN)] = v
            v = plsc.load_gather(xv, [rbase + (_L - _LN)])
            idxb[pl.ds(_L - _LN, _LN)] = v
            d1, d2 = gather_descs(r, emb, idxb, sem)
            d1.start()

            @pl.when(ln > _C2OFF)
            def _():
                d2.start()

        def wait_row(r, emb, idxb, sem, ln):
            d1, d2 = gather_descs(r, emb, idxb, sem)
            d1.wait()

            @pl.when(ln > _C2OFF)
            def _():
                d2.wait()

        def compute_row(r, emb, len_vec):
            ng = (len_vec[0] + (_LN - 1)) >> 4

            def grp_body(g, carry):
                s_acc, t_acc, wt_acc = carry
                tok = g * _LN + iota
                mask = tok < len_vec
                norm = zero16
                p = zero16
                for d0 in range(_D):
                    dcol = (d0 + iota) & (_D - 1)
                    a = plsc.load_gather(emb, [tok, dcol])
                    norm = norm + a * a
                    am = jnp.where(mask, a, 0.0)
                    p = p + wrot[d0] * am
                w = jnp.where(mask, jnp.exp(norm), 0.0)
                return (s_acc + p, t_acc + w * p, wt_acc + w)

            s_acc, t_acc, wt_acc = lax.fori_loop(
                0, ng, grp_body, (zero16, zero16, zero16))
            lenf_vec = len_vec.astype(jnp.float32)
            s_vec = jnp.full((_LN,), jnp.sum(s_acc), jnp.float32)
            t_vec = jnp.full((_LN,), jnp.sum(t_acc), jnp.float32)
            w_vec = jnp.full((_LN,), jnp.sum(wt_acc), jnp.float32)
            score_vec = s_vec / lenf_vec + t_vec / w_vec + b_vec
            plsc.store_scatter(scores, [jnp.full((_LN,), r, jnp.int32)],
                               score_vec, mask=iota == 0)

        for j in range(_DEPTH - 1):
            lv = row_len(j)
            start_row(j, embs[j], idxs[j], sems[j], lv[0])

        def quad_body(q, _):
            r0 = _DEPTH * q
            for j in range(_DEPTH):
                r = r0 + j
                lv = row_len(r)
                wait_row(r, embs[j], idxs[j], sems[j], lv[0])
                rn = r + _DEPTH - 1
                jn = (j + _DEPTH - 1) % _DEPTH

                @pl.when(rn < _ROWS)
                def _(rn=rn, jn=jn):
                    lvn = row_len(rn)
                    start_row(rn, embs[jn], idxs[jn], sems[jn], lvn[0])

                compute_row(r, embs[j], lv)
            return 0

        lax.fori_loop(0, _ROWS // _DEPTH, quad_body, 0)

        for g in range(_ROWS // _LN):
            sv = scores[pl.ds(g * _LN, _LN)]
            scores[pl.ds(g * _LN, _LN)] = 1.0 / (1.0 + jnp.exp(-sv))
        pltpu.sync_copy(scores, out_hbm.at[pl.ds(base, _ROWS)])

    return sc_kernel


_transpose_kernel = _make_transpose_kernel()
_sc_kernel = _make_sc_kernel()


def kernel(X, lens, table, W, b):
    wb = jnp.concatenate(
        [W.reshape(-1).astype(jnp.float32),
         b.reshape(-1).astype(jnp.float32),
         jnp.zeros((48 - _D - 1,), jnp.float32)])
    tail = table[_NFULL:, :].reshape(_TAILW // 4, 4 * _D)
    lin = _transpose_kernel(table.T, tail)
    prob = _sc_kernel(X.astype(jnp.int32).reshape(-1),
                      lens.astype(jnp.int32),
                      lin.reshape(_VOCAB, _D), wb)
    return prob.reshape(_B, 1)


# speedup vs baseline: 1.4966x; 1.0085x over previous
import functools

import jax
import jax.numpy as jnp
from jax import lax
from jax.experimental import pallas as pl
from jax.experimental.pallas import tpu as pltpu
from jax.experimental.pallas import tpu_sc as plsc

_VOCAB = 1000000
_D = 32
_B = 4096
_L = 200

_NC = 2
_NS = 16
_LN = 16
_NW = _NC * _NS
_ROWS = _B // _NW
_NG = (_L + _LN - 1) // _LN
_EMB_ROWS = _NG * _LN
_C1 = 104
_C2OFF = 96
_DEPTH = 4
_QW = 128


_NFULL = _VOCAB // _QW * _QW
_NBLK = _NFULL // _QW
_TAILW = _VOCAB - _NFULL


def _make_transpose_kernel():
    mesh = plsc.VectorSubcoreMesh(core_axis_name="c", subcore_axis_name="s")

    @functools.partial(
        pl.kernel,
        mesh=mesh,
        compiler_params=pltpu.CompilerParams(
            needs_layout_passes=False, use_tc_tiling_on_sc=True),
        out_type=jax.ShapeDtypeStruct((_VOCAB * _D,), jnp.float32),
        scratch_types=[
            [pltpu.VMEM((_D, _QW), jnp.float32) for _ in range(4)],
            [pltpu.VMEM((_D * _QW,), jnp.float32) for _ in range(4)],
            [pltpu.SemaphoreType.DMA for _ in range(4)],
            [pltpu.SemaphoreType.DMA for _ in range(4)],
        ],
    )
    def t_kernel(tabt_hbm, tail_hbm, out_hbm, ibs, obs, sis, sos):
        wid = lax.axis_index("s") * _NC + lax.axis_index("c")
        iota = lax.iota(jnp.int32, _LN)
        iota_q = iota >> 2
        iotalo = (iota & 3) * _D

        def in_descs(k, ib, sem):
            c0 = (wid + _NW * k) * _QW
            return [pltpu.make_async_copy(
                tabt_hbm.at[pl.ds(8 * rr, 8), pl.ds(c0, _QW)],
                ib.at[pl.ds(8 * rr, 8)], sem) for rr in range(4)]

        def blk_ok(k):
            return wid + _NW * k < _NBLK

        def start_in(k, ib, sem):
            @pl.when(blk_ok(k))
            def _():
                for dsc in in_descs(k, ib, sem):
                    dsc.start()

        def wait_in(k, ib, sem):
            @pl.when(blk_ok(k))
            def _():
                for dsc in in_descs(k, ib, sem):
                    dsc.wait()

        def out_desc(k, ob, sem):
            return pltpu.make_async_copy(
                ob, out_hbm.at[pl.ds((wid + _NW * k) * _D * _QW, _D * _QW)],
                sem)

        def transpose_blk(k, ib, ob, sem):
            @pl.when(blk_ok(k))
            def _():
                for d in range(_D):
                    rowv = (d + iota) & (_D - 1)
                    obase = iota * _D + rowv
                    for c0 in range(0, _QW, _LN):
                        val = plsc.load_gather(ib, [rowv, c0 + iota])
                        plsc.store_scatter(ob, [c0 * _D + obase], val)
                out_desc(k, ob, sem).start()

        def wait_out(k, ob, sem):
            @pl.when((k >= 0) & blk_ok(k))
            def _():
                out_desc(k, ob, sem).wait()

        for j in range(3):
            start_in(j, ibs[j], sis[j])

        def quad_body(q, _):
            for j in range(4):
                k = 4 * q + j
                jn = (j + 3) % 4
                start_in(k + 3, ibs[jn], sis[jn])
                wait_in(k, ibs[j], sis[j])
                wait_out(k - 4, obs[j], sos[j])
                transpose_blk(k, ibs[j], obs[j], sos[j])
            return 0

        nquads = (_NBLK // _NW + 4) // 4
        lax.fori_loop(0, nquads, quad_body, 0)
        for j in range(4):
            wait_out(4 * (nquads - 1) + j, obs[j], sos[j])

        @pl.when(wid == 0)
        def _():
            pltpu.sync_copy(tail_hbm, obs[0].at[pl.ds(0, _TAILW * _D)])
            pltpu.sync_copy(obs[0].at[pl.ds(0, _TAILW * _D)],
                            out_hbm.at[pl.ds(_NFULL * _D, _TAILW * _D)])

    return t_kernel


def _make_sc_kernel():
    mesh = plsc.VectorSubcoreMesh(core_axis_name="c", subcore_axis_name="s")

    @functools.partial(
        pl.kernel,
        mesh=mesh,
        compiler_params=pltpu.CompilerParams(
            needs_layout_passes=False, use_tc_tiling_on_sc=False),
        out_type=jax.ShapeDtypeStruct((_B,), jnp.float32),
        scratch_types=[
            pltpu.VMEM((_ROWS * _L,), jnp.int32),
            [pltpu.VMEM((_L + 8, _D), jnp.float32)
             for _ in range(_DEPTH)],
            [pltpu.VMEM((_L + 8,), jnp.int32)
             for _ in range(_DEPTH)],
            pltpu.VMEM((_ROWS,), jnp.int32),
            pltpu.VMEM((48,), jnp.float32),
            pltpu.VMEM((_ROWS,), jnp.float32),
            [pltpu.SemaphoreType.DMA for _ in range(_DEPTH)],
        ],
    )
    def sc_kernel(x_hbm, lens_hbm, table_hbm, wb_hbm, out_hbm,
                  xv, embs, idxs, lens_v, wb_v, scores, sems):
        wid = lax.axis_index("s") * _NC + lax.axis_index("c")
        base = wid * _ROWS

        pltpu.sync_copy(x_hbm.at[pl.ds(base * _L, _ROWS * _L)], xv)
        pltpu.sync_copy(lens_hbm.at[pl.ds(base, _ROWS)], lens_v)
        pltpu.sync_copy(wb_hbm, wb_v)
        wv2 = wb_v[pl.ds(2 * _LN, _LN)]
        b_vec = jnp.full((_LN,), wv2[0], jnp.float32)
        iota = lax.iota(jnp.int32, _LN)
        zero16 = jnp.zeros((_LN,), jnp.float32)
        wrot = [plsc.load_gather(wb_v, [(d0 + iota) & (_D - 1)])
                for d0 in range(_D)]

        def row_len(r):
            return plsc.load_gather(lens_v, [jnp.full((_LN,), r, jnp.int32)])

        def gather_descs(r, emb, idxb, sem):
            d1 = pltpu.make_async_copy(
                table_hbm.at[idxb.at[pl.ds(0, _C1)]],
                emb.at[pl.ds(0, _C1)], sem)
            d2 = pltpu.make_async_copy(
                table_hbm.at[idxb.at[pl.ds(_C2OFF, _C1)]],
                emb.at[pl.ds(_C2OFF, _C1)], sem)
            return d1, d2

        def start_row(r, emb, idxb, sem, ln):
            rbase = r * _L + iota
            for g in range(12):
                off = g * _LN
                v = plsc.load_gather(xv, [rbase + off])
                idxb[pl.ds(off, _LN)] = v
            v = plsc.load_gather(xv, [rbase + (_L - _LN)])
            idxb[pl.ds(_L - _LN, _LN)] = v
            d1, d2 = gather_descs(r, emb, idxb, sem)
            d1.start()

            @pl.when(ln > _C2OFF)
            def _():
                d2.start()

        def wait_row(r, emb, idxb, sem, ln):
            d1, d2 = gather_descs(r, emb, idxb, sem)
            d1.wait()

            @pl.when(ln > _C2OFF)
            def _():
                d2.wait()

        def compute_row(r, emb, len_vec):
            ng = (len_vec[0] + (_LN - 1)) >> 4

            def grp_body(g, carry):
                s_acc, t_acc, wt_acc = carry
                tok = g * _LN + iota
                mask = tok < len_vec
                norm = zero16
                p = zero16
                for d0 in range(_D):
                    dcol = (d0 + iota) & (_D - 1)
                    a = plsc.load_gather(emb, [tok, dcol])
                    norm = norm + a * a
                    am = jnp.where(mask, a, 0.0)
                    p = p + wrot[d0] * am
                w = jnp.where(mask, jnp.exp(norm), 0.0)
                return (s_acc + p, t_acc + w * p, wt_acc + w)

            s_acc, t_acc, wt_acc = lax.fori_loop(
                0, ng, grp_body, (zero16, zero16, zero16))
            lenf_vec = len_vec.astype(jnp.float32)
            s_vec = jnp.full((_LN,), jnp.sum(s_acc), jnp.float32)
            t_vec = jnp.full((_LN,), jnp.sum(t_acc), jnp.float32)
            w_vec = jnp.full((_LN,), jnp.sum(wt_acc), jnp.float32)
            score_vec = s_vec / lenf_vec + t_vec / w_vec + b_vec
            plsc.store_scatter(scores, [jnp.full((_LN,), r, jnp.int32)],
                               score_vec, mask=iota == 0)

        for j in range(_DEPTH - 1):
            lv = row_len(j)
            start_row(j, embs[j], idxs[j], sems[j], lv[0])

        def quad_body(q, _):
            r0 = _DEPTH * q
            for j in range(_DEPTH):
                r = r0 + j
                lv = row_len(r)
                wait_row(r, embs[j], idxs[j], sems[j], lv[0])
                rn = r + _DEPTH - 1
                jn = (j + _DEPTH - 1) % _DEPTH

                @pl.when(rn < _ROWS)
                def _(rn=rn, jn=jn):
                    lvn = row_len(rn)
                    start_row(rn, embs[jn], idxs[jn], sems[jn], lvn[0])

                compute_row(r, embs[j], lv)
            return 0

        lax.fori_loop(0, _ROWS // _DEPTH, quad_body, 0)

        for g in range(_ROWS // _LN):
            sv = scores[pl.ds(g * _LN, _LN)]
            scores[pl.ds(g * _LN, _LN)] = 1.0 / (1.0 + jnp.exp(-sv))
        pltpu.sync_copy(scores, out_hbm.at[pl.ds(base, _ROWS)])

    return sc_kernel


_transpose_kernel = _make_transpose_kernel()
_sc_kernel = _make_sc_kernel()


def kernel(X, lens, table, W, b):
    wb = jnp.concatenate(
        [W.reshape(-1).astype(jnp.float32),
         b.reshape(-1).astype(jnp.float32),
         jnp.zeros((48 - _D - 1,), jnp.float32)])
    tail = table[_NFULL:, :].reshape(-1)
    lin = _transpose_kernel(table.T, tail)
    prob = _sc_kernel(X.astype(jnp.int32).reshape(-1),
                      lens.astype(jnp.int32),
                      lin.reshape(_VOCAB, _D), wb)
    return prob.reshape(_B, 1)
